# baseline (speedup 1.0000x reference)
import functools

import jax
import jax.numpy as jnp
from jax import lax
from jax.experimental import pallas as pl
from jax.experimental.pallas import tpu as pltpu
from jax.experimental.pallas import tpu_sc as plsc

N_NODES = 10000
VOCAB = 21128
EMB_DIM = 128
OUT_DIM = 10
L = 16
NCORE = 2
NSUB = 16
NP = 10240
RPW = NP // NSUB
NYC = RPW // 128
NB = 16

_SC_PARAMS = pltpu.CompilerParams(use_tc_tiling_on_sc=False,
                                  needs_layout_passes=False)

_MESH = dict(core_axis_name="c", subcore_axis_name="s",
             num_cores=NCORE, num_subcores=NSUB)


def _tc_matmul(emb, w16):
    def body(a_ref, w_ref, o_ref):
        o_ref[...] = jnp.dot(a_ref[...], w_ref[...],
                             preferred_element_type=jnp.float32)
    return pl.pallas_call(
        body,
        out_shape=jax.ShapeDtypeStruct((VOCAB, L), jnp.float32),
    )(emb, w16)


def _rsqrt16(x):
    i = plsc.bitcast(x, jnp.int32)
    h = plsc.bitcast(jnp.full((L,), 0x5F3759DF, jnp.int32)
                     - (i >> jnp.full((L,), 1, jnp.int32)), jnp.float32)
    for _ in range(3):
        h = h * (1.5 - 0.5 * x * h * h)
    return h


def _sc_degree(adj_u, adj_r, epr, n_chunks):
    epw = n_chunks * 128

    @functools.partial(
        pl.kernel,
        out_type=jax.ShapeDtypeStruct((NCORE, NP), jnp.float32),
        mesh=plsc.VectorSubcoreMesh(**_MESH),
        scratch_types=[pltpu.VMEM_SHARED((NP,), jnp.float32),
                       pltpu.VMEM((epw,), jnp.int32),
                       pltpu.VMEM((RPW,), jnp.float32),
                       pltpu.SemaphoreType.DMA,
                       pltpu.SemaphoreType.DMA],
        compiler_params=_SC_PARAMS,
    )
    def k(adju_hbm, adjr_hbm, deg_hbm, deg_sh, dst_v, ones_v, ssem, hsem):
        cid = lax.axis_index("c")
        sid = lax.axis_index("s")
        base = sid * RPW
        ebase = sid * epr

        @pl.when(cid == 0)
        def _():
            pltpu.async_copy(adju_hbm.at[1].at[pl.ds(ebase, epr)],
                             dst_v.at[pl.ds(0, epr)], hsem)

        @pl.when(cid == 1)
        def _():
            pltpu.async_copy(adjr_hbm.at[1].at[pl.ds(ebase, epr)],
                             dst_v.at[pl.ds(0, epr)], hsem)

        @pl.loop(0, RPW, step=L)
        def _(i):
            ones_v[pl.ds(i, L)] = jnp.full((L,), 1.0, jnp.float32)

        @pl.loop(epr, epw, step=L)
        def _(i):
            dst_v[pl.ds(i, L)] = jnp.full((L,), NP - 1, jnp.int32)

        pltpu.make_async_copy(adju_hbm.at[1].at[pl.ds(ebase, epr)],
                              dst_v.at[pl.ds(0, epr)], hsem).wait()
        pltpu.sync_copy(ones_v, deg_sh.at[pl.ds(base, RPW)])
        plsc.subcore_barrier()

        @pl.loop(0, n_chunks, step=NB)
        def _(j0):
            dd = [pltpu.async_copy(ones_v.at[pl.ds(0, 128)],
                                   deg_sh.at[dst_v.at[pl.ds((j0 + t) * 128,
                                                            128)]],
                                   ssem, add=True)
                  for t in range(NB)]
            for d in dd:
                d.wait()
        plsc.subcore_barrier()

        pltpu.sync_copy(deg_sh.at[pl.ds(base, RPW)], ones_v)
        pltpu.sync_copy(ones_v, deg_hbm.at[cid, pl.ds(base, RPW)])

    return k(adj_u, adj_r)


def _sc_main(tw, uidx, ridx, adj_u, adj_r, deg, b16, epr, n_chunks):
    epw = n_chunks * 128

    @functools.partial(
        pl.kernel,
        out_type=jax.ShapeDtypeStruct((NCORE, NP, L), jnp.float32),
        mesh=plsc.VectorSubcoreMesh(**_MESH),
        scratch_types=[pltpu.VMEM_SHARED((NP, L), jnp.float32),
                       pltpu.VMEM_SHARED((NP, L), jnp.float32),
                       pltpu.VMEM((RPW,), jnp.int32),
                       pltpu.VMEM((epw,), jnp.int32),
                       pltpu.VMEM((epw,), jnp.int32),
                       pltpu.VMEM((RPW, L), jnp.float32),
                       pltpu.VMEM((RPW, L), jnp.float32),
                       pltpu.VMEM((RPW, L), jnp.float32),
                       pltpu.VMEM((NB, 128, L), jnp.float32),
                       pltpu.VMEM((RPW,), jnp.float32),
                       pltpu.VMEM((RPW,), jnp.float32),
                       pltpu.VMEM((L,), jnp.float32),
                       pltpu.SemaphoreType.DMA,
                       pltpu.SemaphoreType.DMA,
                       pltpu.SemaphoreType.DMA],
        compiler_params=_SC_PARAMS,
    )
    def k(tw_hbm, uidx_hbm, ridx_hbm, adju_hbm, adjr_hbm, deg_hbm, b_hbm,
          out_hbm, z_sh, acc_sh,
          nidx_v, src_v, dst_v, yrows_v, stage_v, dinv16_v, rows_v,
          deg_v, dinv_v, b_v, gsem, ssem, hsem):
        cid = lax.axis_index("c")
        sid = lax.axis_index("s")
        base = sid * RPW
        ebase = sid * epr

        @pl.when(cid == 0)
        def _():
            pltpu.async_copy(uidx_hbm.at[pl.ds(base, RPW)], nidx_v, hsem)
            pltpu.async_copy(adju_hbm.at[0].at[pl.ds(ebase, epr)],
                             src_v.at[pl.ds(0, epr)], hsem)
            pltpu.async_copy(adju_hbm.at[1].at[pl.ds(ebase, epr)],
                             dst_v.at[pl.ds(0, epr)], hsem)

        @pl.when(cid == 1)
        def _():
            pltpu.async_copy(ridx_hbm.at[pl.ds(base, RPW)], nidx_v, hsem)
            pltpu.async_copy(adjr_hbm.at[0].at[pl.ds(ebase, epr)],
                             src_v.at[pl.ds(0, epr)], hsem)
            pltpu.async_copy(adjr_hbm.at[1].at[pl.ds(ebase, epr)],
                             dst_v.at[pl.ds(0, epr)], hsem)
        pltpu.async_copy(deg_hbm.at[cid, pl.ds(base, RPW)], deg_v, hsem)
        pltpu.async_copy(b_hbm, b_v, hsem)

        @pl.loop(0, RPW)
        def _(i):
            stage_v.at[i][...] = jnp.zeros((L,), jnp.float32)

        @pl.loop(epr, epw, step=L)
        def _(i):
            src_v[pl.ds(i, L)] = jnp.full((L,), NP - 1, jnp.int32)
            dst_v[pl.ds(i, L)] = jnp.full((L,), NP - 1, jnp.int32)

        pltpu.sync_copy(stage_v, acc_sh.at[pl.ds(base, RPW)])

        pltpu.make_async_copy(uidx_hbm.at[pl.ds(base, RPW)],
                              nidx_v, hsem).wait()
        gd = [pltpu.async_copy(tw_hbm.at[nidx_v.at[pl.ds(j * 128, 128)]],
                               yrows_v.at[pl.ds(j * 128, 128)], gsem)
              for j in range(NYC)]

        pltpu.make_async_copy(deg_hbm.at[cid, pl.ds(base, RPW)],
                              deg_v, hsem).wait()

        @pl.loop(0, RPW // L, unroll=8)
        def _(i):
            dinv_v[pl.ds(i * L, L)] = _rsqrt16(deg_v[pl.ds(i * L, L)])

        for d in gd:
            d.wait()

        @pl.loop(0, RPW, unroll=8)
        def _(r):
            dv = plsc.load_gather(dinv_v, [jnp.full((L,), r, jnp.int32)])
            dinv16_v.at[r][...] = dv
            yrows_v.at[r][...] = yrows_v.at[r][...] * dv

        pltpu.sync_copy(yrows_v, z_sh.at[pl.ds(base, RPW)])

        pltpu.make_async_copy(adju_hbm.at[0].at[pl.ds(ebase, epr)],
                              src_v.at[pl.ds(0, epr)], hsem).wait()
        pltpu.make_async_copy(adju_hbm.at[1].at[pl.ds(ebase, epr)],
                              dst_v.at[pl.ds(0, epr)], hsem).wait()
        pltpu.make_async_copy(b_hbm, b_v, hsem).wait()
        plsc.subcore_barrier()

        HB = NB // 2
        ngr = n_chunks // HB

        def fire_gathers(g, bank):
            return [pltpu.async_copy(
                z_sh.at[src_v.at[pl.ds((g * HB + t) * 128, 128)]],
                rows_v.at[bank * HB + t], gsem) for t in range(HB)]

        def fire_scatters(g, bank, gds):
            sds = []
            for t in range(HB):
                gds[t].wait()
                sds.append(pltpu.async_copy(
                    rows_v.at[bank * HB + t],
                    acc_sh.at[dst_v.at[pl.ds((g * HB + t) * 128, 128)]],
                    ssem, add=True))
            return sds

        gd0 = fire_gathers(0, 0)
        sd_prev = fire_scatters(0, 0, gd0)
        gd_cur = fire_gathers(1, 1)
        for g in range(1, ngr):
            bank = g % 2
            sd_new = fire_scatters(g, bank, gd_cur)
            for d in sd_prev:
                d.wait()
            if g + 1 < ngr:
                gd_cur = fire_gathers(g + 1, (g + 1) % 2)
            sd_prev = sd_new
        for d in sd_prev:
            d.wait()
        plsc.subcore_barrier()

        pltpu.sync_copy(acc_sh.at[pl.ds(base, RPW)], stage_v)
        bvec = b_v[...]

        @pl.loop(0, RPW, unroll=8)
        def _(r):
            stage_v.at[r][...] = (dinv16_v.at[r][...]
                                  * (stage_v.at[r][...] + yrows_v.at[r][...])
                                  + bvec)
        pltpu.sync_copy(stage_v, out_hbm.at[cid, pl.ds(base, RPW)])

    return k(tw, uidx, ridx, adj_u, adj_r, deg, b16)


def kernel(utterance_input, response_input, utterance_graph_adj,
           response_graph_adj, emb_table, W, b):
    e = utterance_graph_adj.shape[1]
    epr = e // NSUB
    gran = 128 * NB
    n_chunks = ((epr + gran - 1) // gran) * NB

    npad = NP - N_NODES
    uidx = jnp.concatenate([utterance_input.astype(jnp.int32),
                            jnp.zeros((npad,), jnp.int32)])
    ridx = jnp.concatenate([response_input.astype(jnp.int32),
                            jnp.zeros((npad,), jnp.int32)])
    w16 = jnp.pad(W, ((0, 0), (0, L - OUT_DIM)))
    b16 = jnp.pad(b, (0, L - OUT_DIM))

    deg = _sc_degree(utterance_graph_adj, response_graph_adj, epr, n_chunks)
    tw = _tc_matmul(emb_table, w16)
    out = _sc_main(tw, uidx, ridx, utterance_graph_adj, response_graph_adj,
                   deg, b16, epr, n_chunks)
    return (out[0, :N_NODES, :OUT_DIM], out[1, :N_NODES, :OUT_DIM])


# speedup vs baseline: 1.0065x; 1.0065x over previous
import functools

import jax
import jax.numpy as jnp
from jax import lax
from jax.experimental import pallas as pl
from jax.experimental.pallas import tpu as pltpu
from jax.experimental.pallas import tpu_sc as plsc

N_NODES = 10000
VOCAB = 21128
EMB_DIM = 128
OUT_DIM = 10
L = 16
NCORE = 2
NSUB = 16
NP = 10240
RPW = NP // NSUB
NYC = RPW // 128
NB = 16

_SC_PARAMS = pltpu.CompilerParams(use_tc_tiling_on_sc=False,
                                  needs_layout_passes=False)

_MESH = dict(core_axis_name="c", subcore_axis_name="s",
             num_cores=NCORE, num_subcores=NSUB)


def _tc_matmul(emb, w16):
    def body(a_ref, w_ref, o_ref):
        o_ref[...] = jnp.dot(a_ref[...], w_ref[...],
                             preferred_element_type=jnp.float32)
    return pl.pallas_call(
        body,
        out_shape=jax.ShapeDtypeStruct((VOCAB, L), jnp.float32),
    )(emb, w16)


def _rsqrt16(x):
    i = plsc.bitcast(x, jnp.int32)
    h = plsc.bitcast(jnp.full((L,), 0x5F3759DF, jnp.int32)
                     - (i >> jnp.full((L,), 1, jnp.int32)), jnp.float32)
    for _ in range(3):
        h = h * (1.5 - 0.5 * x * h * h)
    return h


def _sc_degree(adj_u, adj_r, epr, n_chunks):
    epw = n_chunks * 128

    @functools.partial(
        pl.kernel,
        out_type=jax.ShapeDtypeStruct((NCORE, NP), jnp.float32),
        mesh=plsc.VectorSubcoreMesh(**_MESH),
        scratch_types=[pltpu.VMEM_SHARED((NP,), jnp.float32),
                       pltpu.VMEM((epw,), jnp.int32),
                       pltpu.VMEM((RPW,), jnp.float32),
                       pltpu.SemaphoreType.DMA,
                       pltpu.SemaphoreType.DMA],
        compiler_params=_SC_PARAMS,
    )
    def k(adju_hbm, adjr_hbm, deg_hbm, deg_sh, dst_v, ones_v, ssem, hsem):
        cid = lax.axis_index("c")
        sid = lax.axis_index("s")
        base = sid * RPW
        ebase = sid * epr

        @pl.when(cid == 0)
        def _():
            pltpu.async_copy(adju_hbm.at[1].at[pl.ds(ebase, epr)],
                             dst_v.at[pl.ds(0, epr)], hsem)

        @pl.when(cid == 1)
        def _():
            pltpu.async_copy(adjr_hbm.at[1].at[pl.ds(ebase, epr)],
                             dst_v.at[pl.ds(0, epr)], hsem)

        @pl.loop(0, RPW, step=L)
        def _(i):
            ones_v[pl.ds(i, L)] = jnp.full((L,), 1.0, jnp.float32)

        @pl.loop(epr, epw, step=L)
        def _(i):
            dst_v[pl.ds(i, L)] = jnp.full((L,), NP - 1, jnp.int32)

        pltpu.make_async_copy(adju_hbm.at[1].at[pl.ds(ebase, epr)],
                              dst_v.at[pl.ds(0, epr)], hsem).wait()
        pltpu.sync_copy(ones_v, deg_sh.at[pl.ds(base, RPW)])
        plsc.subcore_barrier()

        @pl.loop(0, n_chunks, step=NB)
        def _(j0):
            dd = [pltpu.async_copy(ones_v.at[pl.ds(0, 128)],
                                   deg_sh.at[dst_v.at[pl.ds((j0 + t) * 128,
                                                            128)]],
                                   ssem, add=True)
                  for t in range(NB)]
            for d in dd:
                d.wait()
        plsc.subcore_barrier()

        pltpu.sync_copy(deg_sh.at[pl.ds(base, RPW)], ones_v)
        pltpu.sync_copy(ones_v, deg_hbm.at[cid, pl.ds(base, RPW)])

    return k(adj_u, adj_r)


def _sc_main(tw, uidx, ridx, adj_u, adj_r, deg, b16, epr, n_chunks):
    epw = n_chunks * 128

    @functools.partial(
        pl.kernel,
        out_type=jax.ShapeDtypeStruct((NCORE, NP, L), jnp.float32),
        mesh=plsc.VectorSubcoreMesh(**_MESH),
        scratch_types=[pltpu.VMEM_SHARED((NP, L), jnp.float32),
                       pltpu.VMEM_SHARED((NP, L), jnp.float32),
                       pltpu.VMEM((RPW,), jnp.int32),
                       pltpu.VMEM((epw,), jnp.int32),
                       pltpu.VMEM((epw,), jnp.int32),
                       pltpu.VMEM((RPW, L), jnp.float32),
                       pltpu.VMEM((RPW, L), jnp.float32),
                       pltpu.VMEM((RPW, L), jnp.float32),
                       pltpu.VMEM((NB, 128, L), jnp.float32),
                       pltpu.VMEM((RPW,), jnp.float32),
                       pltpu.VMEM((RPW,), jnp.float32),
                       pltpu.VMEM((L,), jnp.float32),
                       pltpu.SemaphoreType.DMA,
                       pltpu.SemaphoreType.DMA,
                       pltpu.SemaphoreType.DMA],
        compiler_params=_SC_PARAMS,
    )
    def k(tw_hbm, uidx_hbm, ridx_hbm, adju_hbm, adjr_hbm, deg_hbm, b_hbm,
          out_hbm, z_sh, acc_sh,
          nidx_v, src_v, dst_v, yrows_v, stage_v, dinv16_v, rows_v,
          deg_v, dinv_v, b_v, gsem, ssem, hsem):
        cid = lax.axis_index("c")
        sid = lax.axis_index("s")
        base = sid * RPW
        ebase = sid * epr

        @pl.when(cid == 0)
        def _():
            pltpu.async_copy(uidx_hbm.at[pl.ds(base, RPW)], nidx_v, hsem)
            pltpu.async_copy(adju_hbm.at[0].at[pl.ds(ebase, epr)],
                             src_v.at[pl.ds(0, epr)], hsem)
            pltpu.async_copy(adju_hbm.at[1].at[pl.ds(ebase, epr)],
                             dst_v.at[pl.ds(0, epr)], hsem)

        @pl.when(cid == 1)
        def _():
            pltpu.async_copy(ridx_hbm.at[pl.ds(base, RPW)], nidx_v, hsem)
            pltpu.async_copy(adjr_hbm.at[0].at[pl.ds(ebase, epr)],
                             src_v.at[pl.ds(0, epr)], hsem)
            pltpu.async_copy(adjr_hbm.at[1].at[pl.ds(ebase, epr)],
                             dst_v.at[pl.ds(0, epr)], hsem)
        pltpu.async_copy(deg_hbm.at[cid, pl.ds(base, RPW)], deg_v, hsem)
        pltpu.async_copy(b_hbm, b_v, hsem)

        @pl.loop(0, RPW)
        def _(i):
            stage_v.at[i][...] = jnp.zeros((L,), jnp.float32)

        @pl.loop(epr, epw, step=L)
        def _(i):
            src_v[pl.ds(i, L)] = jnp.full((L,), NP - 1, jnp.int32)
            dst_v[pl.ds(i, L)] = jnp.full((L,), NP - 1, jnp.int32)

        pltpu.sync_copy(stage_v, acc_sh.at[pl.ds(base, RPW)])

        pltpu.make_async_copy(uidx_hbm.at[pl.ds(base, RPW)],
                              nidx_v, hsem).wait()
        gd = [pltpu.async_copy(tw_hbm.at[nidx_v.at[pl.ds(j * 128, 128)]],
                               yrows_v.at[pl.ds(j * 128, 128)], gsem)
              for j in range(NYC)]

        pltpu.make_async_copy(deg_hbm.at[cid, pl.ds(base, RPW)],
                              deg_v, hsem).wait()

        @pl.loop(0, RPW // L, unroll=4)
        def _(i):
            dinv_v[pl.ds(i * L, L)] = _rsqrt16(deg_v[pl.ds(i * L, L)])

        for d in gd:
            d.wait()

        @pl.loop(0, RPW, unroll=4)
        def _(r):
            dv = plsc.load_gather(dinv_v, [jnp.full((L,), r, jnp.int32)])
            dinv16_v.at[r][...] = dv
            yrows_v.at[r][...] = yrows_v.at[r][...] * dv

        pltpu.sync_copy(yrows_v, z_sh.at[pl.ds(base, RPW)])

        pltpu.make_async_copy(adju_hbm.at[0].at[pl.ds(ebase, epr)],
                              src_v.at[pl.ds(0, epr)], hsem).wait()
        pltpu.make_async_copy(adju_hbm.at[1].at[pl.ds(ebase, epr)],
                              dst_v.at[pl.ds(0, epr)], hsem).wait()
        pltpu.make_async_copy(b_hbm, b_v, hsem).wait()
        plsc.subcore_barrier()

        HB = NB // 2
        ngr = n_chunks // HB

        def fire_gathers(g, bank):
            return [pltpu.async_copy(
                z_sh.at[src_v.at[pl.ds((g * HB + t) * 128, 128)]],
                rows_v.at[bank * HB + t], gsem) for t in range(HB)]

        def fire_scatters(g, bank, gds):
            sds = []
            for t in range(HB):
                gds[t].wait()
                sds.append(pltpu.async_copy(
                    rows_v.at[bank * HB + t],
                    acc_sh.at[dst_v.at[pl.ds((g * HB + t) * 128, 128)]],
                    ssem, add=True))
            return sds

        gd0 = fire_gathers(0, 0)
        sd_prev = fire_scatters(0, 0, gd0)
        gd_cur = fire_gathers(1, 1)
        for g in range(1, ngr):
            bank = g % 2
            sd_new = fire_scatters(g, bank, gd_cur)
            for d in sd_prev:
                d.wait()
            if g + 1 < ngr:
                gd_cur = fire_gathers(g + 1, (g + 1) % 2)
            sd_prev = sd_new
        for d in sd_prev:
            d.wait()
        plsc.subcore_barrier()

        pltpu.sync_copy(acc_sh.at[pl.ds(base, RPW)], stage_v)
        bvec = b_v[...]

        @pl.loop(0, RPW, unroll=4)
        def _(r):
            stage_v.at[r][...] = (dinv16_v.at[r][...]
                                  * (stage_v.at[r][...] + yrows_v.at[r][...])
                                  + bvec)
        pltpu.sync_copy(stage_v, out_hbm.at[cid, pl.ds(base, RPW)])

    return k(tw, uidx, ridx, adj_u, adj_r, deg, b16)


def kernel(utterance_input, response_input, utterance_graph_adj,
           response_graph_adj, emb_table, W, b):
    e = utterance_graph_adj.shape[1]
    epr = e // NSUB
    gran = 128 * NB
    n_chunks = ((epr + gran - 1) // gran) * NB

    npad = NP - N_NODES
    uidx = jnp.concatenate([utterance_input.astype(jnp.int32),
                            jnp.zeros((npad,), jnp.int32)])
    ridx = jnp.concatenate([response_input.astype(jnp.int32),
                            jnp.zeros((npad,), jnp.int32)])
    w16 = jnp.pad(W, ((0, 0), (0, L - OUT_DIM)))
    b16 = jnp.pad(b, (0, L - OUT_DIM))

    deg = _sc_degree(utterance_graph_adj, response_graph_adj, epr, n_chunks)
    tw = _tc_matmul(emb_table, w16)
    out = _sc_main(tw, uidx, ridx, utterance_graph_adj, response_graph_adj,
                   deg, b16, epr, n_chunks)
    return (out[0, :N_NODES, :OUT_DIM], out[1, :N_NODES, :OUT_DIM])
